# baseline (speedup 1.0000x reference)
import functools

import jax
import jax.numpy as jnp
from jax import lax
from jax.experimental import pallas as pl
from jax.experimental.pallas import tpu as pltpu
from jax.experimental.pallas import tpu_sc as plsc


@functools.cache
def _make_embed_gather(V, D, B):
    info = plsc.get_sparse_core_info()
    NC, NS = info.num_cores, info.num_subcores
    NW = NC * NS
    assert B % NW == 0
    b_per_w = B // NW
    CH = 16
    assert b_per_w % (2 * CH) == 0 and CH <= 128
    NCH = b_per_w // CH
    R = NCH // 2

    mesh = plsc.VectorSubcoreMesh(core_axis_name="c", subcore_axis_name="s")

    @functools.partial(
        pl.kernel,
        mesh=mesh,
        out_type=jax.ShapeDtypeStruct((B, D), jnp.float32),
        scratch_types=[
            pltpu.VMEM((NCH, CH), jnp.int32),
            pltpu.VMEM((CH, D), jnp.float32),
            pltpu.VMEM((CH, D), jnp.float32),
        ]
        + [pltpu.SemaphoreType.DMA] * 4,
    )
    def k(idx_hbm, table_hbm, out_hbm, idx_v, buf0, buf1, gs0, gs1, ws0, ws1):
        wid = lax.axis_index("s") * NC + lax.axis_index("c")
        base = wid * b_per_w
        pltpu.sync_copy(idx_hbm.at[pl.ds(wid * NCH, NCH)], idx_v)

        def gather(c, buf, sem):
            pltpu.async_copy(table_hbm.at[idx_v.at[c]], buf, sem)

        def gather_wait(buf, sem):
            pltpu.make_async_copy(out_hbm.at[pl.ds(0, CH)], buf, sem).wait()

        def write(c, buf, sem):
            off = pl.multiple_of(base + c * CH, 8)
            pltpu.async_copy(buf, out_hbm.at[pl.ds(off, CH)], sem)

        def write_wait(buf, sem):
            pltpu.make_async_copy(buf, out_hbm.at[pl.ds(0, CH)], sem).wait()

        gather(0, buf0, gs0)
        gather(1, buf1, gs1)

        def body(r, carry):
            c0 = 2 * r
            gather_wait(buf0, gs0)
            write(c0, buf0, ws0)
            gather_wait(buf1, gs1)
            write(c0 + 1, buf1, ws1)

            @pl.when(r < R - 1)
            def _():
                write_wait(buf0, ws0)
                gather(c0 + 2, buf0, gs0)
                write_wait(buf1, ws1)
                gather(c0 + 3, buf1, gs1)

            return carry

        lax.fori_loop(0, R, body, 0)
        write_wait(buf0, ws0)
        write_wait(buf1, ws1)

    def run(idx_flat, table):
        return k(idx_flat.reshape(NW * NCH, CH), table)

    return run


def kernel(tokens, W_E):
    B, P = tokens.shape
    V, D = W_E.shape
    idx = tokens.reshape(-1).astype(jnp.int32)
    out = _make_embed_gather(V, D, B * P)(idx, W_E)
    return out.reshape(B, P, D)


# speedup vs baseline: 1.1157x; 1.1157x over previous
import functools

import jax
import jax.numpy as jnp
from jax import lax
from jax.experimental import pallas as pl
from jax.experimental.pallas import tpu as pltpu
from jax.experimental.pallas import tpu_sc as plsc


@functools.cache
def _make_embed_gather(V, D, BT, PT):
    B = BT * PT
    info = plsc.get_sparse_core_info()
    NC, NS = info.num_cores, info.num_subcores
    NW = NC * NS
    assert B % NW == 0
    b_per_w = B // NW
    assert PT % b_per_w == 0
    CH = 32
    NBUF = 4
    assert b_per_w % CH == 0 and CH <= 128
    NCH = b_per_w // CH

    mesh = plsc.VectorSubcoreMesh(core_axis_name="c", subcore_axis_name="s")

    @functools.partial(
        pl.kernel,
        mesh=mesh,
        out_type=jax.ShapeDtypeStruct((B, D), jnp.float32),
        scratch_types=[
            pltpu.VMEM((b_per_w,), jnp.int32),
        ]
        + [pltpu.VMEM((CH, D), jnp.float32)] * NBUF
        + [pltpu.SemaphoreType.DMA] * (2 * NBUF),
    )
    def k(idx_hbm, table_hbm, out_hbm, idx_v, *rest):
        bufs = rest[:NBUF]
        gsems = rest[NBUF : 2 * NBUF]
        wsems = rest[2 * NBUF :]

        wid = lax.axis_index("s") * NC + lax.axis_index("c")
        base = wid * b_per_w
        row = wid // (PT // b_per_w)
        col = (wid % (PT // b_per_w)) * b_per_w
        pltpu.sync_copy(idx_hbm.at[row, pl.ds(col, b_per_w)], idx_v)

        def gather(c):
            s = c % NBUF
            return pltpu.async_copy(
                table_hbm.at[idx_v.at[pl.ds(c * CH, CH)]], bufs[s], gsems[s]
            )

        K = NBUF - 1
        gathers = [None] * NCH
        writes = [None] * NCH
        for c in range(min(K, NCH)):
            gathers[c] = gather(c)
        for c in range(NCH):
            s = c % NBUF
            n = c + K
            if n < NCH:
                if n - NBUF >= 0:
                    writes[n - NBUF].wait()
                gathers[n] = gather(n)
            gathers[c].wait()
            writes[c] = pltpu.async_copy(
                bufs[s], out_hbm.at[pl.ds(base + c * CH, CH)], wsems[s]
            )
        for c in range(max(0, NCH - NBUF), NCH):
            writes[c].wait()

    return k


def kernel(tokens, W_E):
    BT, PT = tokens.shape
    V, D = W_E.shape
    out = _make_embed_gather(V, D, BT, PT)(tokens.astype(jnp.int32), W_E)
    return out.reshape(BT, PT, D)


# speedup vs baseline: 1.1315x; 1.0142x over previous
import functools

import jax
import jax.numpy as jnp
from jax import lax
from jax.experimental import pallas as pl
from jax.experimental.pallas import tpu as pltpu
from jax.experimental.pallas import tpu_sc as plsc


@functools.cache
def _make_embed_gather(V, D, BT, PT):
    B = BT * PT
    info = plsc.get_sparse_core_info()
    NC, NS = info.num_cores, info.num_subcores
    NW = NC * NS
    assert B % NW == 0
    b_per_w = B // NW
    assert PT % b_per_w == 0
    CH = 32
    NBUF = 5
    assert b_per_w % CH == 0 and CH <= 128
    NCH = b_per_w // CH

    mesh = plsc.VectorSubcoreMesh(core_axis_name="c", subcore_axis_name="s")

    @functools.partial(
        pl.kernel,
        mesh=mesh,
        out_type=jax.ShapeDtypeStruct((B, D), jnp.float32),
        scratch_types=[
            pltpu.VMEM((b_per_w,), jnp.int32),
        ]
        + [pltpu.VMEM((CH, D), jnp.float32)] * NBUF
        + [pltpu.SemaphoreType.DMA] * (2 * NBUF),
    )
    def k(idx_hbm, table_hbm, out_hbm, idx_v, *rest):
        bufs = rest[:NBUF]
        gsems = rest[NBUF : 2 * NBUF]
        wsems = rest[2 * NBUF :]

        wid = lax.axis_index("s") * NC + lax.axis_index("c")
        base = wid * b_per_w
        row = wid // (PT // b_per_w)
        col = (wid % (PT // b_per_w)) * b_per_w
        pltpu.sync_copy(idx_hbm.at[row, pl.ds(col, b_per_w)], idx_v)

        def gather(c):
            s = c % NBUF
            return pltpu.async_copy(
                table_hbm.at[idx_v.at[pl.ds(c * CH, CH)]], bufs[s], gsems[s]
            )

        K = NBUF - 1
        gathers = [None] * NCH
        writes = [None] * NCH
        for c in range(min(K, NCH)):
            gathers[c] = gather(c)
        for c in range(NCH):
            s = c % NBUF
            n = c + K
            if n < NCH:
                if n - NBUF >= 0:
                    writes[n - NBUF].wait()
                gathers[n] = gather(n)
            gathers[c].wait()
            writes[c] = pltpu.async_copy(
                bufs[s], out_hbm.at[pl.ds(base + c * CH, CH)], wsems[s]
            )
        for c in range(max(0, NCH - NBUF), NCH):
            writes[c].wait()

    return k


def kernel(tokens, W_E):
    BT, PT = tokens.shape
    V, D = W_E.shape
    out = _make_embed_gather(V, D, BT, PT)(tokens.astype(jnp.int32), W_E)
    return out.reshape(BT, PT, D)
